# baseline (speedup 1.0000x reference)
import functools

import jax
import jax.numpy as jnp
from jax import lax
from jax.experimental import pallas as pl
from jax.experimental.pallas import tpu as pltpu
from jax.experimental.pallas import tpu_sc as plsc

N = 100_000
E = 6_400_000
LANES = 16
CHUNK = 8000
VPC = CHUNK // LANES
NC = 2
NS = 16
NW = NC * NS
CPW = E // CHUNK // NW
SLICE = 6256
LAST_SLICE = N - (NS - 1) * SLICE

_mesh = plsc.VectorSubcoreMesh(
    core_axis_name="c", subcore_axis_name="s", num_cores=NC, num_subcores=NS
)


@functools.partial(
    pl.kernel,
    out_type=jax.ShapeDtypeStruct((NC * N,), jnp.float32),
    mesh=_mesh,
    scratch_types=[
        pltpu.VMEM((CHUNK,), jnp.int32),
        pltpu.VMEM((CHUNK,), jnp.int32),
        pltpu.VMEM((CHUNK,), jnp.int32),
        pltpu.VMEM((CHUNK,), jnp.float32),
        pltpu.VMEM((CHUNK,), jnp.float32),
        pltpu.VMEM((CHUNK,), jnp.float32),
        pltpu.VMEM((CHUNK,), jnp.int32),
        pltpu.VMEM((CHUNK,), jnp.int32),
        pltpu.VMEM((CHUNK,), jnp.int32),
        pltpu.VMEM((CHUNK,), jnp.int32),
        pltpu.VMEM((CHUNK,), jnp.float32),
        pltpu.VMEM((CHUNK,), jnp.float32),
        pltpu.VMEM((CHUNK,), jnp.float32),
        pltpu.VMEM((CHUNK,), jnp.float32),
        pltpu.VMEM_SHARED((N,), jnp.float32),
        pltpu.VMEM_SHARED((N,), jnp.float32),
        pltpu.SemaphoreType.DMA,
        pltpu.SemaphoreType.DMA,
        pltpu.SemaphoreType.DMA,
    ],
    compiler_params=pltpu.CompilerParams(needs_layout_passes=False),
)
def _spmv_sc(ed_hbm, row_hbm, col_hbm, b_hbm, out_hbm,
             col_v0, col_v1, col_v2, ed_v0, ed_v1, ed_v2,
             row_v0, row_v1, row_v2, row_v3,
             bval_v0, bval_v1, prod_v0, prod_v1,
             b_sh, acc, sem_in, sem_g, sem_sc):
    col_v = (col_v0, col_v1, col_v2)
    ed_v = (ed_v0, ed_v1, ed_v2)
    row_v = (row_v0, row_v1, row_v2, row_v3)
    bval_v = (bval_v0, bval_v1)
    prod_v = (prod_v0, prod_v1)
    c = lax.axis_index("c")
    s = lax.axis_index("s")
    wid = s * NC + c

    @plsc.parallel_loop(0, SLICE // LANES + 1, unroll=4)
    def _(k):
        prod_v0[pl.ds(k * LANES, LANES)] = jnp.zeros((LANES,), jnp.float32)

    @pl.when(s < NS - 1)
    def _():
        pltpu.sync_copy(b_hbm.at[pl.ds(s * SLICE, SLICE)],
                        bval_v0.at[pl.ds(0, SLICE)])
        pltpu.sync_copy(bval_v0.at[pl.ds(0, SLICE)],
                        b_sh.at[pl.ds(s * SLICE, SLICE)])
        pltpu.sync_copy(prod_v0.at[pl.ds(0, SLICE)],
                        acc.at[pl.ds(s * SLICE, SLICE)])

    @pl.when(s == NS - 1)
    def _():
        pltpu.sync_copy(b_hbm.at[pl.ds((NS - 1) * SLICE, LAST_SLICE)],
                        bval_v0.at[pl.ds(0, LAST_SLICE)])
        pltpu.sync_copy(bval_v0.at[pl.ds(0, LAST_SLICE)],
                        b_sh.at[pl.ds((NS - 1) * SLICE, LAST_SLICE)])
        pltpu.sync_copy(prod_v0.at[pl.ds(0, LAST_SLICE)],
                        acc.at[pl.ds((NS - 1) * SLICE, LAST_SLICE)])

    def fire_loads(i):
        e0 = (i * NW + wid) * CHUNK
        sl = pl.ds(e0, CHUNK)
        pltpu.async_copy(col_hbm.at[sl], col_v[i % 3], sem_in)
        pltpu.async_copy(ed_hbm.at[sl], ed_v[i % 3], sem_in)
        pltpu.async_copy(row_hbm.at[sl], row_v[i % 4], sem_in)

    def wait_loads(i):
        sl = pl.ds(0, CHUNK)
        pltpu.make_async_copy(col_hbm.at[sl], col_v[i % 3], sem_in).wait()
        pltpu.make_async_copy(ed_hbm.at[sl], ed_v[i % 3], sem_in).wait()
        pltpu.make_async_copy(row_hbm.at[sl], row_v[i % 4], sem_in).wait()

    def fire_gather(i):
        pltpu.async_copy(b_sh.at[col_v[i % 3]], bval_v[i % 2], sem_g)

    def wait_gather():
        pltpu.make_async_copy(ed_hbm.at[pl.ds(0, CHUNK)],
                              prod_v0, sem_g).wait()

    def drain_scatter():
        pltpu.make_async_copy(ed_hbm.at[pl.ds(0, CHUNK)],
                              prod_v0, sem_sc).wait()

    def compute_scatter(i):
        bv, pv, ev = bval_v[i % 2], prod_v[i % 2], ed_v[i % 3]

        @plsc.parallel_loop(0, VPC, unroll=8)
        def _(k):
            sl = pl.ds(k * LANES, LANES)
            pv[sl] = ev[sl] * bv[sl]

        pltpu.async_copy(pv, acc.at[row_v[i % 4]], sem_sc, add=True)

    fire_loads(0)
    fire_loads(1)
    plsc.subcore_barrier()
    wait_loads(0)
    fire_gather(0)

    for i in range(CPW):
        wait_gather()
        if i >= 2:
            drain_scatter()
        if i + 2 < CPW:
            fire_loads(i + 2)
        if i + 1 < CPW:
            wait_loads(i + 1)
            fire_gather(i + 1)
        compute_scatter(i)

    drain_scatter()
    drain_scatter()

    plsc.subcore_barrier()

    @pl.when(s < NS - 1)
    def _():
        pltpu.sync_copy(acc.at[pl.ds(s * SLICE, SLICE)],
                        prod_v0.at[pl.ds(0, SLICE)])
        pltpu.sync_copy(prod_v0.at[pl.ds(0, SLICE)],
                        out_hbm.at[pl.ds(c * N + s * SLICE, SLICE)])

    @pl.when(s == NS - 1)
    def _():
        pltpu.sync_copy(acc.at[pl.ds((NS - 1) * SLICE, LAST_SLICE)],
                        prod_v0.at[pl.ds(0, LAST_SLICE)])
        pltpu.sync_copy(
            prod_v0.at[pl.ds(0, LAST_SLICE)],
            out_hbm.at[pl.ds(c * N + (NS - 1) * SLICE, LAST_SLICE)])


def _combine_body(p_ref, o_ref):
    o_ref[...] = p_ref[0:1, :] + p_ref[1:2, :]


def kernel(edata, row, col, B):
    partial = _spmv_sc(edata, row, col, B).reshape(NC, N)
    out = pl.pallas_call(
        _combine_body,
        out_shape=jax.ShapeDtypeStruct((1, N), jnp.float32),
    )(partial)
    return out.reshape(N)


# speedup vs baseline: 1.1410x; 1.1410x over previous
import functools

import jax
import jax.numpy as jnp
from jax import lax
from jax.experimental import pallas as pl
from jax.experimental.pallas import tpu as pltpu
from jax.experimental.pallas import tpu_sc as plsc

N = 100_000
E = 6_400_000
LANES = 16
CHUNK = 1600
VPC = CHUNK // LANES
NC = 2
NS = 16
NW = NC * NS
CPW = E // CHUNK // NW
UNROLL = 4
STEPS = 30
TAIL = CPW - STEPS * UNROLL
SLICE = 6256
LAST_SLICE = N - (NS - 1) * SLICE
PIECE = CHUNK

_mesh = plsc.VectorSubcoreMesh(
    core_axis_name="c", subcore_axis_name="s", num_cores=NC, num_subcores=NS
)


@functools.partial(
    pl.kernel,
    out_type=jax.ShapeDtypeStruct((NC * N,), jnp.float32),
    mesh=_mesh,
    scratch_types=[
        pltpu.VMEM((N,), jnp.float32),
        pltpu.VMEM((CHUNK,), jnp.int32),
        pltpu.VMEM((CHUNK,), jnp.int32),
        pltpu.VMEM((CHUNK,), jnp.int32),
        pltpu.VMEM((CHUNK,), jnp.int32),
        pltpu.VMEM((CHUNK,), jnp.float32),
        pltpu.VMEM((CHUNK,), jnp.float32),
        pltpu.VMEM((CHUNK,), jnp.float32),
        pltpu.VMEM((CHUNK,), jnp.float32),
        pltpu.VMEM((CHUNK,), jnp.int32),
        pltpu.VMEM((CHUNK,), jnp.int32),
        pltpu.VMEM((CHUNK,), jnp.int32),
        pltpu.VMEM((CHUNK,), jnp.int32),
        pltpu.VMEM((CHUNK,), jnp.float32),
        pltpu.VMEM((CHUNK,), jnp.float32),
        pltpu.VMEM_SHARED((N,), jnp.float32),
        pltpu.SemaphoreType.DMA,
        pltpu.SemaphoreType.DMA,
    ],
    compiler_params=pltpu.CompilerParams(needs_layout_passes=False),
)
def _spmv_sc(ed_hbm, row_hbm, col_hbm, b_hbm, out_hbm,
             b_v, col_v0, col_v1, col_v2, col_v3,
             ed_v0, ed_v1, ed_v2, ed_v3,
             row_v0, row_v1, row_v2, row_v3,
             prod_v0, prod_v1,
             acc, sem_in, sem_sc):
    col_v = (col_v0, col_v1, col_v2, col_v3)
    ed_v = (ed_v0, ed_v1, ed_v2, ed_v3)
    row_v = (row_v0, row_v1, row_v2, row_v3)
    prod_v = (prod_v0, prod_v1)
    c = lax.axis_index("c")
    s = lax.axis_index("s")
    wid = s * NC + c

    def zero_body(k, carry):
        prod_v0[pl.ds(k * LANES, LANES)] = jnp.zeros((LANES,), jnp.float32)
        return carry

    lax.fori_loop(0, PIECE // LANES, zero_body, 0)

    @pl.when(s < NS - 1)
    def _():
        for p0 in range(0, SLICE, PIECE):
            w = min(PIECE, SLICE - p0)
            pltpu.sync_copy(prod_v0.at[pl.ds(0, w)],
                            acc.at[pl.ds(s * SLICE + p0, w)])

    @pl.when(s == NS - 1)
    def _():
        for p0 in range(0, LAST_SLICE, PIECE):
            w = min(PIECE, LAST_SLICE - p0)
            pltpu.sync_copy(prod_v0.at[pl.ds(0, w)],
                            acc.at[pl.ds((NS - 1) * SLICE + p0, w)])

    def fire_loads(i, b4):
        e0 = (i * NW + wid) * CHUNK
        sl = pl.ds(e0, CHUNK)
        pltpu.async_copy(col_hbm.at[sl], col_v[b4], sem_in)
        pltpu.async_copy(ed_hbm.at[sl], ed_v[b4], sem_in)
        pltpu.async_copy(row_hbm.at[sl], row_v[b4], sem_in)

    def wait_loads():
        pltpu.make_async_copy(ed_hbm.at[pl.ds(0, 3 * CHUNK)],
                              b_v.at[pl.ds(0, 3 * CHUNK)], sem_in).wait()

    def drain_scatter():
        pltpu.make_async_copy(ed_hbm.at[pl.ds(0, CHUNK)],
                              prod_v0, sem_sc).wait()

    def chunk_body(i, q, drain, fire_ahead):
        wait_loads()
        if drain:
            drain_scatter()
        if fire_ahead:
            fire_loads(i + 3, (q + 3) % 4)
        b4, b2 = q % 4, q % 2

        @plsc.parallel_loop(0, VPC, unroll=8)
        def _(k):
            sl = pl.ds(k * LANES, LANES)
            bvals = plsc.load_gather(b_v, [col_v[b4][sl]])
            prod_v[b2][sl] = ed_v[b4][sl] * bvals

        pltpu.async_copy(prod_v[b2], acc.at[row_v[b4]], sem_sc, add=True)

    pltpu.async_copy(b_hbm, b_v, sem_sc)
    fire_loads(0, 0)
    fire_loads(1, 1)
    fire_loads(2, 2)
    plsc.subcore_barrier()
    pltpu.make_async_copy(b_hbm, b_v, sem_sc).wait()

    for q in range(UNROLL):
        chunk_body(q, q, drain=q >= 1, fire_ahead=True)

    def step_body(p, carry):
        base = p * UNROLL
        for q in range(UNROLL):
            chunk_body(base + q, q, drain=True, fire_ahead=True)
        return carry

    lax.fori_loop(1, STEPS, step_body, 0)

    for q in range(TAIL):
        chunk_body(STEPS * UNROLL + q, q, drain=True, fire_ahead=q < TAIL - 3)
    drain_scatter()

    plsc.subcore_barrier()

    @pl.when(s < NS - 1)
    def _():
        for p0 in range(0, SLICE, PIECE):
            w = min(PIECE, SLICE - p0)
            pltpu.sync_copy(acc.at[pl.ds(s * SLICE + p0, w)],
                            prod_v0.at[pl.ds(0, w)])
            pltpu.sync_copy(prod_v0.at[pl.ds(0, w)],
                            out_hbm.at[pl.ds(c * N + s * SLICE + p0, w)])

    @pl.when(s == NS - 1)
    def _():
        for p0 in range(0, LAST_SLICE, PIECE):
            w = min(PIECE, LAST_SLICE - p0)
            pltpu.sync_copy(acc.at[pl.ds((NS - 1) * SLICE + p0, w)],
                            prod_v0.at[pl.ds(0, w)])
            pltpu.sync_copy(
                prod_v0.at[pl.ds(0, w)],
                out_hbm.at[pl.ds(c * N + (NS - 1) * SLICE + p0, w)])


def _combine_body(p_ref, o_ref):
    o_ref[...] = p_ref[0:1, :] + p_ref[1:2, :]


def kernel(edata, row, col, B):
    partial = _spmv_sc(edata, row, col, B).reshape(NC, N)
    out = pl.pallas_call(
        _combine_body,
        out_shape=jax.ShapeDtypeStruct((1, N), jnp.float32),
    )(partial)
    return out.reshape(N)


# speedup vs baseline: 1.5574x; 1.3650x over previous
import functools

import jax
import jax.numpy as jnp
from jax import lax
from jax.experimental import pallas as pl
from jax.experimental.pallas import tpu as pltpu
from jax.experimental.pallas import tpu_sc as plsc

N = 100_000
E = 6_400_000
LANES = 16
CHUNK = 1600
VPC = CHUNK // LANES
NC = 2
NS = 16
NW = NC * NS
CPW = E // CHUNK // NW
UNROLL = 4
STEPS = 30
TAIL = CPW - STEPS * UNROLL
SLICE = 6256
LAST_SLICE = N - (NS - 1) * SLICE
PIECE = CHUNK

_mesh = plsc.VectorSubcoreMesh(
    core_axis_name="c", subcore_axis_name="s", num_cores=NC, num_subcores=NS
)


@functools.partial(
    pl.kernel,
    out_type=jax.ShapeDtypeStruct((NC * N,), jnp.float32),
    mesh=_mesh,
    scratch_types=[
        pltpu.VMEM((N,), jnp.float32),
        pltpu.VMEM((CHUNK,), jnp.int32),
        pltpu.VMEM((CHUNK,), jnp.int32),
        pltpu.VMEM((CHUNK,), jnp.int32),
        pltpu.VMEM((CHUNK,), jnp.int32),
        pltpu.VMEM((CHUNK,), jnp.float32),
        pltpu.VMEM((CHUNK,), jnp.float32),
        pltpu.VMEM((CHUNK,), jnp.float32),
        pltpu.VMEM((CHUNK,), jnp.float32),
        pltpu.VMEM((CHUNK,), jnp.int32),
        pltpu.VMEM((CHUNK,), jnp.int32),
        pltpu.VMEM((CHUNK,), jnp.int32),
        pltpu.VMEM((CHUNK,), jnp.int32),
        pltpu.VMEM((CHUNK,), jnp.float32),
        pltpu.VMEM((CHUNK,), jnp.float32),
        pltpu.VMEM_SHARED((N,), jnp.float32),
        pltpu.SemaphoreType.DMA,
        pltpu.SemaphoreType.DMA,
        pltpu.SemaphoreType.DMA,
    ],
    compiler_params=pltpu.CompilerParams(needs_layout_passes=False),
)
def _spmv_sc(ed_hbm, row_hbm, col_hbm, b_hbm, out_hbm,
             b_v, col_v0, col_v1, col_v2, col_v3,
             ed_v0, ed_v1, ed_v2, ed_v3,
             row_v0, row_v1, row_v2, row_v3,
             prod_v0, prod_v1,
             acc, sem_in, sem_row, sem_sc):
    col_v = (col_v0, col_v1, col_v2, col_v3)
    ed_v = (ed_v0, ed_v1, ed_v2, ed_v3)
    row_v = (row_v0, row_v1, row_v2, row_v3)
    prod_v = (prod_v0, prod_v1)
    c = lax.axis_index("c")
    s = lax.axis_index("s")
    wid = s * NC + c

    def zero_body(k, carry):
        prod_v0[pl.ds(k * LANES, LANES)] = jnp.zeros((LANES,), jnp.float32)
        return carry

    lax.fori_loop(0, PIECE // LANES, zero_body, 0)

    @pl.when(s < NS - 1)
    def _():
        for p0 in range(0, SLICE, PIECE):
            w = min(PIECE, SLICE - p0)
            pltpu.sync_copy(prod_v0.at[pl.ds(0, w)],
                            acc.at[pl.ds(s * SLICE + p0, w)])

    @pl.when(s == NS - 1)
    def _():
        for p0 in range(0, LAST_SLICE, PIECE):
            w = min(PIECE, LAST_SLICE - p0)
            pltpu.sync_copy(prod_v0.at[pl.ds(0, w)],
                            acc.at[pl.ds((NS - 1) * SLICE + p0, w)])

    def fire_col_ed(i, b4):
        e0 = (i * NW + wid) * CHUNK
        sl = pl.ds(e0, CHUNK)
        pltpu.async_copy(col_hbm.at[sl], col_v[b4], sem_in)
        pltpu.async_copy(ed_hbm.at[sl], ed_v[b4], sem_in)

    def fire_row(i, b4):
        e0 = (i * NW + wid) * CHUNK
        pltpu.async_copy(row_hbm.at[pl.ds(e0, CHUNK)], row_v[b4], sem_row)

    def wait_loads():
        pltpu.make_async_copy(ed_hbm.at[pl.ds(0, 2 * CHUNK)],
                              b_v.at[pl.ds(0, 2 * CHUNK)], sem_in).wait()
        pltpu.make_async_copy(ed_hbm.at[pl.ds(0, CHUNK)],
                              prod_v1, sem_row).wait()

    def drain_scatter():
        pltpu.make_async_copy(ed_hbm.at[pl.ds(0, CHUNK)],
                              prod_v0, sem_sc).wait()

    def chunk_body(i, q, drain, fire3, fire2):
        wait_loads()
        if drain:
            drain_scatter()
        if fire3:
            fire_col_ed(i + 3, (q + 3) % 4)
        if fire2:
            fire_row(i + 2, (q + 2) % 4)
        b4, b2 = q % 4, q % 2

        @plsc.parallel_loop(0, VPC, unroll=8)
        def _(k):
            sl = pl.ds(k * LANES, LANES)
            bvals = plsc.load_gather(b_v, [col_v[b4][sl]])
            prod_v[b2][sl] = ed_v[b4][sl] * bvals

        pltpu.async_copy(prod_v[b2], acc.at[row_v[b4]], sem_sc, add=True)

    pltpu.async_copy(b_hbm, b_v, sem_sc)
    fire_col_ed(0, 0)
    fire_col_ed(1, 1)
    fire_col_ed(2, 2)
    fire_row(0, 0)
    fire_row(1, 1)
    plsc.subcore_barrier()
    pltpu.make_async_copy(b_hbm, b_v, sem_sc).wait()

    for q in range(UNROLL):
        chunk_body(q, q, drain=q >= 2, fire3=True, fire2=True)

    def step_body(p, carry):
        base = p * UNROLL
        for q in range(UNROLL):
            chunk_body(base + q, q, drain=True, fire3=True, fire2=True)
        return carry

    lax.fori_loop(1, STEPS, step_body, 0)

    for q in range(TAIL):
        chunk_body(STEPS * UNROLL + q, q, drain=True,
                   fire3=q < TAIL - 3, fire2=q < TAIL - 2)
    drain_scatter()
    drain_scatter()

    plsc.subcore_barrier()

    @pl.when(s < NS - 1)
    def _():
        for p0 in range(0, SLICE, PIECE):
            w = min(PIECE, SLICE - p0)
            pltpu.sync_copy(acc.at[pl.ds(s * SLICE + p0, w)],
                            prod_v0.at[pl.ds(0, w)])
            pltpu.sync_copy(prod_v0.at[pl.ds(0, w)],
                            out_hbm.at[pl.ds(c * N + s * SLICE + p0, w)])

    @pl.when(s == NS - 1)
    def _():
        for p0 in range(0, LAST_SLICE, PIECE):
            w = min(PIECE, LAST_SLICE - p0)
            pltpu.sync_copy(acc.at[pl.ds((NS - 1) * SLICE + p0, w)],
                            prod_v0.at[pl.ds(0, w)])
            pltpu.sync_copy(
                prod_v0.at[pl.ds(0, w)],
                out_hbm.at[pl.ds(c * N + (NS - 1) * SLICE + p0, w)])


def _combine_body(p_ref, o_ref):
    o_ref[...] = p_ref[0:1, :] + p_ref[1:2, :]


def kernel(edata, row, col, B):
    partial = _spmv_sc(edata, row, col, B).reshape(NC, N)
    out = pl.pallas_call(
        _combine_body,
        out_shape=jax.ShapeDtypeStruct((1, N), jnp.float32),
    )(partial)
    return out.reshape(N)


# speedup vs baseline: 1.6116x; 1.0348x over previous
import functools

import jax
import jax.numpy as jnp
from jax import lax
from jax.experimental import pallas as pl
from jax.experimental.pallas import tpu as pltpu
from jax.experimental.pallas import tpu_sc as plsc

N = 100_000
E = 6_400_000
LANES = 16
CHUNK = 1600
VPC = CHUNK // LANES
NC = 2
NS = 16
NW = NC * NS
CPW = E // CHUNK // NW
UNROLL = 4
STEPS = 30
TAIL = CPW - STEPS * UNROLL
SLICE = 6256
LAST_SLICE = N - (NS - 1) * SLICE
PIECE = CHUNK

_mesh = plsc.VectorSubcoreMesh(
    core_axis_name="c", subcore_axis_name="s", num_cores=NC, num_subcores=NS
)


@functools.partial(
    pl.kernel,
    out_type=jax.ShapeDtypeStruct((NC * N,), jnp.float32),
    mesh=_mesh,
    scratch_types=[
        pltpu.VMEM((N,), jnp.float32),
        pltpu.VMEM((CHUNK,), jnp.int32),
        pltpu.VMEM((CHUNK,), jnp.int32),
        pltpu.VMEM((CHUNK,), jnp.int32),
        pltpu.VMEM((CHUNK,), jnp.int32),
        pltpu.VMEM((CHUNK,), jnp.float32),
        pltpu.VMEM((CHUNK,), jnp.float32),
        pltpu.VMEM((CHUNK,), jnp.float32),
        pltpu.VMEM((CHUNK,), jnp.float32),
        pltpu.VMEM((CHUNK,), jnp.int32),
        pltpu.VMEM((CHUNK,), jnp.int32),
        pltpu.VMEM((CHUNK,), jnp.int32),
        pltpu.VMEM((CHUNK,), jnp.int32),
        pltpu.VMEM((CHUNK,), jnp.float32),
        pltpu.VMEM((CHUNK,), jnp.float32),
        pltpu.VMEM_SHARED((N,), jnp.float32),
        pltpu.SemaphoreType.DMA,
        pltpu.SemaphoreType.DMA,
        pltpu.SemaphoreType.DMA,
    ],
    compiler_params=pltpu.CompilerParams(needs_layout_passes=False),
)
def _spmv_sc(ed_hbm, row_hbm, col_hbm, b_hbm, out_hbm,
             b_v, col_v0, col_v1, col_v2, col_v3,
             ed_v0, ed_v1, ed_v2, ed_v3,
             row_v0, row_v1, row_v2, row_v3,
             prod_v0, prod_v1,
             acc, sem_in, sem_row, sem_sc):
    col_v = (col_v0, col_v1, col_v2, col_v3)
    ed_v = (ed_v0, ed_v1, ed_v2, ed_v3)
    row_v = (row_v0, row_v1, row_v2, row_v3)
    prod_v = (prod_v0, prod_v1)
    c = lax.axis_index("c")
    s = lax.axis_index("s")
    wid = s * NC + c

    @pl.when(s < NS - 1)
    def _():
        for p0 in range(0, SLICE, PIECE):
            w = min(PIECE, SLICE - p0)
            pltpu.sync_copy(b_hbm.at[pl.ds(s * SLICE + p0, w)],
                            prod_v0.at[pl.ds(0, w)])
            pltpu.sync_copy(prod_v0.at[pl.ds(0, w)],
                            acc.at[pl.ds(s * SLICE + p0, w)])

    @pl.when(s == NS - 1)
    def _():
        for p0 in range(0, LAST_SLICE, PIECE):
            w = min(PIECE, LAST_SLICE - p0)
            pltpu.sync_copy(b_hbm.at[pl.ds((NS - 1) * SLICE + p0, w)],
                            prod_v0.at[pl.ds(0, w)])
            pltpu.sync_copy(prod_v0.at[pl.ds(0, w)],
                            acc.at[pl.ds((NS - 1) * SLICE + p0, w)])

    plsc.subcore_barrier()
    pltpu.sync_copy(acc, b_v)
    plsc.subcore_barrier()

    def zero_body(k, carry):
        prod_v0[pl.ds(k * LANES, LANES)] = jnp.zeros((LANES,), jnp.float32)
        return carry

    lax.fori_loop(0, PIECE // LANES, zero_body, 0)

    @pl.when(s < NS - 1)
    def _():
        for p0 in range(0, SLICE, PIECE):
            w = min(PIECE, SLICE - p0)
            pltpu.sync_copy(prod_v0.at[pl.ds(0, w)],
                            acc.at[pl.ds(s * SLICE + p0, w)])

    @pl.when(s == NS - 1)
    def _():
        for p0 in range(0, LAST_SLICE, PIECE):
            w = min(PIECE, LAST_SLICE - p0)
            pltpu.sync_copy(prod_v0.at[pl.ds(0, w)],
                            acc.at[pl.ds((NS - 1) * SLICE + p0, w)])

    def fire_col_ed(i, b4):
        e0 = (i * NW + wid) * CHUNK
        sl = pl.ds(e0, CHUNK)
        pltpu.async_copy(col_hbm.at[sl], col_v[b4], sem_in)
        pltpu.async_copy(ed_hbm.at[sl], ed_v[b4], sem_in)

    def fire_row(i, b4):
        e0 = (i * NW + wid) * CHUNK
        pltpu.async_copy(row_hbm.at[pl.ds(e0, CHUNK)], row_v[b4], sem_row)

    def wait_loads():
        pltpu.make_async_copy(ed_hbm.at[pl.ds(0, 2 * CHUNK)],
                              b_v.at[pl.ds(0, 2 * CHUNK)], sem_in).wait()
        pltpu.make_async_copy(ed_hbm.at[pl.ds(0, CHUNK)],
                              prod_v1, sem_row).wait()

    def drain_scatter():
        pltpu.make_async_copy(ed_hbm.at[pl.ds(0, CHUNK)],
                              prod_v0, sem_sc).wait()

    def chunk_body(i, q, drain, fire3, fire2):
        wait_loads()
        if drain:
            drain_scatter()
        if fire3:
            fire_col_ed(i + 3, (q + 3) % 4)
        if fire2:
            fire_row(i + 2, (q + 2) % 4)
        b4, b2 = q % 4, q % 2

        @plsc.parallel_loop(0, VPC, unroll=8)
        def _(k):
            sl = pl.ds(k * LANES, LANES)
            bvals = plsc.load_gather(b_v, [col_v[b4][sl]])
            prod_v[b2][sl] = ed_v[b4][sl] * bvals

        pltpu.async_copy(prod_v[b2], acc.at[row_v[b4]], sem_sc, add=True)

    fire_col_ed(0, 0)
    fire_col_ed(1, 1)
    fire_col_ed(2, 2)
    fire_row(0, 0)
    fire_row(1, 1)
    plsc.subcore_barrier()

    for q in range(UNROLL):
        chunk_body(q, q, drain=q >= 2, fire3=True, fire2=True)

    def step_body(p, carry):
        base = p * UNROLL
        for q in range(UNROLL):
            chunk_body(base + q, q, drain=True, fire3=True, fire2=True)
        return carry

    lax.fori_loop(1, STEPS, step_body, 0)

    for q in range(TAIL):
        chunk_body(STEPS * UNROLL + q, q, drain=True,
                   fire3=q < TAIL - 3, fire2=q < TAIL - 2)
    drain_scatter()
    drain_scatter()

    plsc.subcore_barrier()

    @pl.when(s < NS - 1)
    def _():
        for p0 in range(0, SLICE, PIECE):
            w = min(PIECE, SLICE - p0)
            pltpu.sync_copy(acc.at[pl.ds(s * SLICE + p0, w)],
                            prod_v0.at[pl.ds(0, w)])
            pltpu.sync_copy(prod_v0.at[pl.ds(0, w)],
                            out_hbm.at[pl.ds(c * N + s * SLICE + p0, w)])

    @pl.when(s == NS - 1)
    def _():
        for p0 in range(0, LAST_SLICE, PIECE):
            w = min(PIECE, LAST_SLICE - p0)
            pltpu.sync_copy(acc.at[pl.ds((NS - 1) * SLICE + p0, w)],
                            prod_v0.at[pl.ds(0, w)])
            pltpu.sync_copy(
                prod_v0.at[pl.ds(0, w)],
                out_hbm.at[pl.ds(c * N + (NS - 1) * SLICE + p0, w)])


def _combine_body(p_ref, o_ref):
    o_ref[...] = p_ref[0:1, :] + p_ref[1:2, :]


def kernel(edata, row, col, B):
    partial = _spmv_sc(edata, row, col, B).reshape(NC, N)
    out = pl.pallas_call(
        _combine_body,
        out_shape=jax.ShapeDtypeStruct((1, N), jnp.float32),
    )(partial)
    return out.reshape(N)
